# baseline (speedup 1.0000x reference)
import functools

import jax
import jax.numpy as jnp
from jax import lax
from jax.experimental import pallas as pl
from jax.experimental.pallas import tpu as pltpu
from jax.experimental.pallas import tpu_sc as plsc

_FEAT_CNT = [40000] * 26
_F = len(_FEAT_CNT)
_D = 32
_B = 16384
_L = 16

_INFO = plsc.get_sparse_core_info()
_NC, _NS = _INFO.num_cores, _INFO.num_subcores
_NW = _NC * _NS
_PER_W = _B // _NW
_CHUNK_E = 64
_NCHUNK = _PER_W // _CHUNK_E
_ROWS = _CHUNK_E * _F
_IDX_W = 128
_IDX_H = _ROWS // _IDX_W

_ACU = [sum(_FEAT_CNT[:f]) for f in range(_F)]

_TROWS = 1040000
_PLANE = 1 << 18

_mesh = plsc.VectorSubcoreMesh(core_axis_name="c", subcore_axis_name="s")


@functools.partial(
    pl.kernel,
    out_type=jax.ShapeDtypeStruct((_B, _D), jnp.float32),
    mesh=_mesh,
    scratch_types=[
        pltpu.VMEM((_F, _CHUNK_E), jnp.int32),
        pltpu.VMEM((_IDX_H, _IDX_W), jnp.int32),
        pltpu.VMEM((_ROWS, _D), jnp.float32),
        pltpu.VMEM((_CHUNK_E, _D), jnp.float32),
        pltpu.SemaphoreType.DMA,
    ],
    compiler_params=pltpu.CompilerParams(use_tc_tiling_on_sc=False),
)
def _embed_pool(featT_hbm, table_hbm, out_hbm,
                feat_v, idx_v, rows_v, out_v, sem):
    wid = lax.axis_index("s") * _NC + lax.axis_index("c")

    @pl.loop(0, _NCHUNK)
    def _chunk(c):
        e_base = wid * _PER_W + c * _CHUNK_E
        pltpu.sync_copy(featT_hbm.at[:, pl.ds(e_base, _CHUNK_E)], feat_v)

        for f in range(_F):
            off = jnp.int32(_ACU[f])
            for k in range(_CHUNK_E // _L):
                p = f * _CHUNK_E + k * _L
                i = feat_v[f, pl.ds(k * _L, _L)] + off
                idx_v[p // _IDX_W, pl.ds(p % _IDX_W, _L)] = (
                    ((i & jnp.int32(_PLANE - 1)) << 2) | (i >> 18)
                )

        copies = []
        for j in range(_IDX_H):
            copies.append(
                pltpu.async_copy(
                    table_hbm.at[idx_v.at[j]],
                    rows_v.at[pl.ds(j * _IDX_W, _IDX_W)],
                    sem,
                )
            )
        for cp in copies:
            cp.wait()

        @pl.loop(0, _CHUNK_E)
        def _elem(e):
            acc0 = rows_v[e, pl.ds(0, _L)]
            acc1 = rows_v[e, pl.ds(_L, _L)]
            for f in range(1, _F):
                acc0 += rows_v[f * _CHUNK_E + e, pl.ds(0, _L)]
                acc1 += rows_v[f * _CHUNK_E + e, pl.ds(_L, _L)]
            scale = jnp.float32(1.0 / _F)
            out_v[e, pl.ds(0, _L)] = jnp.maximum(acc0 * scale, 0.0)
            out_v[e, pl.ds(_L, _L)] = jnp.maximum(acc1 * scale, 0.0)

        pltpu.sync_copy(out_v, out_hbm.at[pl.ds(e_base, _CHUNK_E)])


_TBLK = 2048
_TGRID = _PLANE // _TBLK
_WBLKS = 1040000 // _TBLK


def _detile_body(w_ref, x_ref):
    m_id = pl.program_id(1)
    for m in range(4):
        @pl.when(m_id == m)
        def _():
            x_ref[:, 32 * m:32 * (m + 1)] = w_ref[...].T


def _detile(tableT):
    return pl.pallas_call(
        _detile_body,
        out_shape=jax.ShapeDtypeStruct((_PLANE, 128), jnp.float32),
        grid=(_TGRID, 4),
        in_specs=[
            pl.BlockSpec(
                (_D, _TBLK),
                lambda i, m: (0, jnp.minimum(i + _TGRID * m, _WBLKS)),
            )
        ],
        out_specs=pl.BlockSpec((_TBLK, 128), lambda i, m: (i, 0)),
    )(tableT)


def kernel(input_feat, table):
    table_lin = _detile(table.T).reshape(4 * _PLANE, _D)
    return _embed_pool(input_feat.T, table_lin)


# speedup vs baseline: 2.9373x; 2.9373x over previous
import functools

import jax
import jax.numpy as jnp
from jax import lax
from jax.experimental import pallas as pl
from jax.experimental.pallas import tpu as pltpu
from jax.experimental.pallas import tpu_sc as plsc

_FEAT_CNT = [40000] * 26
_F = len(_FEAT_CNT)
_D = 32
_B = 16384
_L = 16

_INFO = plsc.get_sparse_core_info()
_NC, _NS = _INFO.num_cores, _INFO.num_subcores
_NW = _NC * _NS
_PER_W = _B // _NW
_CHUNK_E = 64
_NCHUNK = _PER_W // _CHUNK_E
_ROWS = _CHUNK_E * _F
_IDX_W = 128
_IDX_H = _ROWS // _IDX_W

_ACU = [sum(_FEAT_CNT[:f]) for f in range(_F)]

_TROWS = 1040000
_PLANE = 1 << 18

_mesh = plsc.VectorSubcoreMesh(core_axis_name="c", subcore_axis_name="s")


@functools.partial(
    pl.kernel,
    out_type=jax.ShapeDtypeStruct((_B, _D), jnp.float32),
    mesh=_mesh,
    scratch_types=[
        pltpu.VMEM((_F, _CHUNK_E), jnp.int32),
        pltpu.VMEM((_IDX_H, _IDX_W), jnp.int32),
        pltpu.VMEM((_ROWS, _D), jnp.float32),
        pltpu.VMEM((_CHUNK_E, _D), jnp.float32),
        pltpu.SemaphoreType.DMA,
    ],
    compiler_params=pltpu.CompilerParams(use_tc_tiling_on_sc=False),
)
def _embed_pool(featT_hbm, table_hbm, out_hbm,
                feat_v, idx_v, rows_v, out_v, sem):
    wid = lax.axis_index("s") * _NC + lax.axis_index("c")

    @pl.loop(0, _NCHUNK)
    def _chunk(c):
        e_base = wid * _PER_W + c * _CHUNK_E
        pltpu.sync_copy(featT_hbm.at[:, pl.ds(e_base, _CHUNK_E)], feat_v)

        for f in range(_F):
            off = jnp.int32(_ACU[f])
            for k in range(_CHUNK_E // _L):
                p = f * _CHUNK_E + k * _L
                i = feat_v[f, pl.ds(k * _L, _L)] + off
                idx_v[p // _IDX_W, pl.ds(p % _IDX_W, _L)] = (
                    ((i & jnp.int32(_PLANE - 1)) << 2) | (i >> 18)
                )

        copies = []
        for j in range(_IDX_H):
            copies.append(
                pltpu.async_copy(
                    table_hbm.at[idx_v.at[j]],
                    rows_v.at[pl.ds(j * _IDX_W, _IDX_W)],
                    sem,
                )
            )
        for cp in copies:
            cp.wait()

        @pl.loop(0, _CHUNK_E)
        def _elem(e):
            acc0 = rows_v[e, pl.ds(0, _L)]
            acc1 = rows_v[e, pl.ds(_L, _L)]
            for f in range(1, _F):
                acc0 += rows_v[f * _CHUNK_E + e, pl.ds(0, _L)]
                acc1 += rows_v[f * _CHUNK_E + e, pl.ds(_L, _L)]
            scale = jnp.float32(1.0 / _F)
            out_v[e, pl.ds(0, _L)] = jnp.maximum(acc0 * scale, 0.0)
            out_v[e, pl.ds(_L, _L)] = jnp.maximum(acc1 * scale, 0.0)

        pltpu.sync_copy(out_v, out_hbm.at[pl.ds(e_base, _CHUNK_E)])


_TBLK = 4096
_TGRID = _PLANE // _TBLK
_WBLKS = 1040000 // _TBLK


def _detile_body(w0_ref, w1_ref, w2_ref, w3_ref, x_ref):
    v = jnp.concatenate(
        [w0_ref[...], w1_ref[...], w2_ref[...], w3_ref[...]], axis=0
    )
    x_ref[...] = v.T


def _detile(tableT):
    specs = [
        pl.BlockSpec(
            (_D, _TBLK),
            lambda i, m=m: (0, jnp.minimum(i + _TGRID * m, _WBLKS)),
        )
        for m in range(4)
    ]
    return pl.pallas_call(
        _detile_body,
        out_shape=jax.ShapeDtypeStruct((_PLANE, 128), jnp.float32),
        grid=(_TGRID,),
        in_specs=specs,
        out_specs=pl.BlockSpec((_TBLK, 128), lambda i: (i, 0)),
    )(tableT, tableT, tableT, tableT)


def kernel(input_feat, table):
    table_lin = _detile(table.T).reshape(4 * _PLANE, _D)
    return _embed_pool(input_feat.T, table_lin)


# speedup vs baseline: 3.5291x; 1.2015x over previous
import functools

import jax
import jax.numpy as jnp
from jax import lax
from jax.experimental import pallas as pl
from jax.experimental.pallas import tpu as pltpu
from jax.experimental.pallas import tpu_sc as plsc

_FEAT_CNT = [40000] * 26
_F = len(_FEAT_CNT)
_D = 32
_B = 16384
_L = 16

_INFO = plsc.get_sparse_core_info()
_NC, _NS = _INFO.num_cores, _INFO.num_subcores
_NW = _NC * _NS
_PER_W = _B // _NW
_CHUNK_E = 64
_NCHUNK = _PER_W // _CHUNK_E
_ROWS = _CHUNK_E * _F
_IDX_W = 128
_IDX_H = _ROWS // _IDX_W

_ACU = [sum(_FEAT_CNT[:f]) for f in range(_F)]

_TROWS = 1040000
_PLANE = 1 << 18

_mesh = plsc.VectorSubcoreMesh(core_axis_name="c", subcore_axis_name="s")


@functools.partial(
    pl.kernel,
    out_type=jax.ShapeDtypeStruct((_B, _D), jnp.float32),
    mesh=_mesh,
    scratch_types=[
        pltpu.VMEM((_F, _CHUNK_E), jnp.int32),
        pltpu.VMEM((2, _IDX_H, _IDX_W), jnp.int32),
        pltpu.VMEM((2, _ROWS, _D), jnp.float32),
        pltpu.VMEM((_CHUNK_E, _D), jnp.float32),
        pltpu.SemaphoreType.DMA,
        pltpu.SemaphoreType.DMA,
    ],
    compiler_params=pltpu.CompilerParams(use_tc_tiling_on_sc=False),
)
def _embed_pool(featT_hbm, table_hbm, out_hbm,
                feat_v, idx_v, rows_v, out_v, sem0, sem1):
    wid = lax.axis_index("s") * _NC + lax.axis_index("c")
    sems = (sem0, sem1)

    def _load_idx_fire(c, buf):
        e_base = wid * _PER_W + c * _CHUNK_E
        pltpu.sync_copy(featT_hbm.at[:, pl.ds(e_base, _CHUNK_E)], feat_v)
        for f in range(_F):
            off = jnp.int32(_ACU[f])
            for k in range(_CHUNK_E // _L):
                pos = f * _CHUNK_E + k * _L
                i = feat_v[f, pl.ds(k * _L, _L)] + off
                idx_v[buf, pos // _IDX_W, pl.ds(pos % _IDX_W, _L)] = (
                    ((i & jnp.int32(_PLANE - 1)) << 2) | (i >> 18)
                )
        for j in range(_IDX_H):
            pltpu.async_copy(
                table_hbm.at[idx_v.at[buf].at[j]],
                rows_v.at[buf].at[pl.ds(j * _IDX_W, _IDX_W)],
                sems[buf],
            )

    def _wait_rows(buf):
        pltpu.make_async_copy(
            table_hbm.at[pl.ds(0, _ROWS)], rows_v.at[buf], sems[buf]
        ).wait()

    def _pool_store(c, buf):
        e_base = wid * _PER_W + c * _CHUNK_E

        @pl.loop(0, _CHUNK_E)
        def _elem(e):
            acc0 = rows_v[buf, e, pl.ds(0, _L)]
            acc1 = rows_v[buf, e, pl.ds(_L, _L)]
            for f in range(1, _F):
                acc0 += rows_v[buf, f * _CHUNK_E + e, pl.ds(0, _L)]
                acc1 += rows_v[buf, f * _CHUNK_E + e, pl.ds(_L, _L)]
            scale = jnp.float32(1.0 / _F)
            out_v[e, pl.ds(0, _L)] = jnp.maximum(acc0 * scale, 0.0)
            out_v[e, pl.ds(_L, _L)] = jnp.maximum(acc1 * scale, 0.0)

        pltpu.sync_copy(out_v, out_hbm.at[pl.ds(e_base, _CHUNK_E)])

    _load_idx_fire(0, 0)
    for c in range(_NCHUNK):
        buf = c % 2
        if c + 1 < _NCHUNK:
            _load_idx_fire(c + 1, 1 - buf)
        _wait_rows(buf)
        _pool_store(c, buf)


_TBLK = 8192
_TGRID = _PLANE // _TBLK
_WBLKS = 1040000 // _TBLK


def _detile_body(w0_ref, w1_ref, w2_ref, w3_ref, x_ref):
    v = jnp.concatenate(
        [w0_ref[...], w1_ref[...], w2_ref[...], w3_ref[...]], axis=0
    )
    x_ref[...] = v.T


def _detile(tableT):
    specs = [
        pl.BlockSpec(
            (_D, _TBLK),
            lambda i, m=m: (0, jnp.minimum(i + _TGRID * m, _WBLKS)),
        )
        for m in range(4)
    ]
    return pl.pallas_call(
        _detile_body,
        out_shape=jax.ShapeDtypeStruct((_PLANE, 128), jnp.float32),
        grid=(_TGRID,),
        in_specs=specs,
        out_specs=pl.BlockSpec((_TBLK, 128), lambda i: (i, 0)),
    )(tableT, tableT, tableT, tableT)


def kernel(input_feat, table):
    table_lin = _detile(table.T).reshape(4 * _PLANE, _D)
    return _embed_pool(input_feat.T, table_lin)


# speedup vs baseline: 3.5523x; 1.0066x over previous
import functools

import jax
import jax.numpy as jnp
from jax import lax
from jax.experimental import pallas as pl
from jax.experimental.pallas import tpu as pltpu
from jax.experimental.pallas import tpu_sc as plsc

_FEAT_CNT = [40000] * 26
_F = len(_FEAT_CNT)
_D = 32
_B = 16384
_L = 16

_INFO = plsc.get_sparse_core_info()
_NC, _NS = _INFO.num_cores, _INFO.num_subcores
_NW = _NC * _NS
_PER_W = _B // _NW
_CHUNK_E = 64
_NCHUNK = _PER_W // _CHUNK_E
_ROWS = _CHUNK_E * _F
_IDX_W = 128
_IDX_H = _ROWS // _IDX_W

_ACU = [sum(_FEAT_CNT[:f]) for f in range(_F)]

_TROWS = 1040000
_PLANE = 1 << 18

_mesh = plsc.VectorSubcoreMesh(core_axis_name="c", subcore_axis_name="s")


@functools.partial(
    pl.kernel,
    out_type=jax.ShapeDtypeStruct((_B, _D), jnp.float32),
    mesh=_mesh,
    scratch_types=[
        pltpu.VMEM((_F, _CHUNK_E), jnp.int32),
        pltpu.VMEM((2, _IDX_H, _IDX_W), jnp.int32),
        pltpu.VMEM((2, _ROWS, _D), jnp.float32),
        pltpu.VMEM((_CHUNK_E, _D), jnp.float32),
        pltpu.SemaphoreType.DMA,
        pltpu.SemaphoreType.DMA,
    ],
    compiler_params=pltpu.CompilerParams(use_tc_tiling_on_sc=False),
)
def _embed_pool(featT_hbm, table_hbm, out_hbm,
                feat_v, idx_v, rows_v, out_v, sem0, sem1):
    wid = lax.axis_index("s") * _NC + lax.axis_index("c")
    sems = (sem0, sem1)

    def _load_idx_fire(c, buf):
        e_base = wid * _PER_W + c * _CHUNK_E
        pltpu.sync_copy(featT_hbm.at[:, pl.ds(e_base, _CHUNK_E)], feat_v)
        for f in range(_F):
            off = jnp.int32(_ACU[f])
            for k in range(_CHUNK_E // _L):
                pos = f * _CHUNK_E + k * _L
                i = feat_v[f, pl.ds(k * _L, _L)] + off
                idx_v[buf, pos // _IDX_W, pl.ds(pos % _IDX_W, _L)] = (
                    ((i & jnp.int32(_PLANE - 1)) << 2) | (i >> 18)
                )
        for j in range(_IDX_H):
            pltpu.async_copy(
                table_hbm.at[idx_v.at[buf].at[j]],
                rows_v.at[buf].at[pl.ds(j * _IDX_W, _IDX_W)],
                sems[buf],
            )

    def _wait_rows(buf):
        pltpu.make_async_copy(
            table_hbm.at[pl.ds(0, _ROWS)], rows_v.at[buf], sems[buf]
        ).wait()

    def _pool_store(c, buf):
        e_base = wid * _PER_W + c * _CHUNK_E

        @pl.loop(0, _CHUNK_E)
        def _elem(e):
            acc0 = rows_v[buf, e, pl.ds(0, _L)]
            acc1 = rows_v[buf, e, pl.ds(_L, _L)]
            for f in range(1, _F):
                acc0 += rows_v[buf, f * _CHUNK_E + e, pl.ds(0, _L)]
                acc1 += rows_v[buf, f * _CHUNK_E + e, pl.ds(_L, _L)]
            scale = jnp.float32(1.0 / _F)
            out_v[e, pl.ds(0, _L)] = jnp.maximum(acc0 * scale, 0.0)
            out_v[e, pl.ds(_L, _L)] = jnp.maximum(acc1 * scale, 0.0)

        pltpu.sync_copy(out_v, out_hbm.at[pl.ds(e_base, _CHUNK_E)])

    _load_idx_fire(0, 0)
    for c in range(_NCHUNK):
        buf = c % 2
        if c + 1 < _NCHUNK:
            _load_idx_fire(c + 1, 1 - buf)
        _wait_rows(buf)
        _pool_store(c, buf)


_TBLK = 16384
_TGRID = _PLANE // _TBLK
_WBLKS = 1040000 // _TBLK


def _detile_body(w0_ref, w1_ref, w2_ref, w3_ref, x_ref):
    v = jnp.concatenate(
        [w0_ref[...], w1_ref[...], w2_ref[...], w3_ref[...]], axis=0
    )
    x_ref[...] = v.T


def _detile(tableT):
    specs = [
        pl.BlockSpec(
            (_D, _TBLK),
            lambda i, m=m: (0, jnp.minimum(i + _TGRID * m, _WBLKS)),
        )
        for m in range(4)
    ]
    return pl.pallas_call(
        _detile_body,
        out_shape=jax.ShapeDtypeStruct((_PLANE, 128), jnp.float32),
        grid=(_TGRID,),
        in_specs=specs,
        out_specs=pl.BlockSpec((_TBLK, 128), lambda i: (i, 0)),
    )(tableT, tableT, tableT, tableT)


def kernel(input_feat, table):
    table_lin = _detile(table.T).reshape(4 * _PLANE, _D)
    return _embed_pool(input_feat.T, table_lin)


# speedup vs baseline: 3.6964x; 1.0406x over previous
import functools

import jax
import jax.numpy as jnp
from jax import lax
from jax.experimental import pallas as pl
from jax.experimental.pallas import tpu as pltpu
from jax.experimental.pallas import tpu_sc as plsc

_FEAT_CNT = [40000] * 26
_F = len(_FEAT_CNT)
_D = 32
_B = 16384
_L = 16

_INFO = plsc.get_sparse_core_info()
_NC, _NS = _INFO.num_cores, _INFO.num_subcores
_NW = _NC * _NS
_PER_W = _B // _NW
_CHUNK_E = 64
_NCHUNK = _PER_W // _CHUNK_E
_ROWS = _CHUNK_E * _F
_IDX_W = 128
_IDX_H = _ROWS // _IDX_W

_ACU = [sum(_FEAT_CNT[:f]) for f in range(_F)]

_TROWS = 1040000
_PLANE = 1 << 18

_mesh = plsc.VectorSubcoreMesh(core_axis_name="c", subcore_axis_name="s")


@functools.partial(
    pl.kernel,
    out_type=jax.ShapeDtypeStruct((_B // 4, 128), jnp.float32),
    mesh=_mesh,
    scratch_types=[
        pltpu.VMEM((_F, _CHUNK_E), jnp.int32),
        pltpu.VMEM((2, _IDX_H, _IDX_W), jnp.int32),
        pltpu.VMEM((2, _ROWS, _D), jnp.float32),
        pltpu.VMEM((_CHUNK_E, _D), jnp.float32),
        pltpu.SemaphoreType.DMA,
        pltpu.SemaphoreType.DMA,
    ],
    compiler_params=pltpu.CompilerParams(use_tc_tiling_on_sc=False),
)
def _embed_pool(featT_hbm, table_hbm, out_hbm,
                feat_v, idx_v, rows_v, out_v, sem0, sem1):
    wid = lax.axis_index("s") * _NC + lax.axis_index("c")
    sems = (sem0, sem1)

    def _load_idx_fire(c, buf):
        e_base = wid * _PER_W + c * _CHUNK_E
        pltpu.sync_copy(featT_hbm.at[:, pl.ds(e_base, _CHUNK_E)], feat_v)
        for f in range(_F):
            off = jnp.int32(_ACU[f])
            for k in range(_CHUNK_E // _L):
                pos = f * _CHUNK_E + k * _L
                i = feat_v[f, pl.ds(k * _L, _L)] + off
                idx_v[buf, pos // _IDX_W, pl.ds(pos % _IDX_W, _L)] = (
                    ((i & jnp.int32(_PLANE - 1)) << 2) | (i >> 18)
                )
        for j in range(_IDX_H):
            pltpu.async_copy(
                table_hbm.at[idx_v.at[buf].at[j]],
                rows_v.at[buf].at[pl.ds(j * _IDX_W, _IDX_W)],
                sems[buf],
            )

    def _wait_rows(buf):
        pltpu.make_async_copy(
            table_hbm.at[pl.ds(0, _ROWS)], rows_v.at[buf], sems[buf]
        ).wait()

    def _pool_store(c, buf):
        e_base = wid * _PER_W + c * _CHUNK_E
        row_base = (wid & 7) * _PER_W + c * _CHUNK_E
        lane0 = (wid >> 3) * _D

        @pl.loop(0, _CHUNK_E)
        def _elem(e):
            acc0 = rows_v[buf, e, pl.ds(0, _L)]
            acc1 = rows_v[buf, e, pl.ds(_L, _L)]
            for f in range(1, _F):
                acc0 += rows_v[buf, f * _CHUNK_E + e, pl.ds(0, _L)]
                acc1 += rows_v[buf, f * _CHUNK_E + e, pl.ds(_L, _L)]
            scale = jnp.float32(1.0 / _F)
            out_v[e, pl.ds(0, _L)] = jnp.maximum(acc0 * scale, 0.0)
            out_v[e, pl.ds(_L, _L)] = jnp.maximum(acc1 * scale, 0.0)

        pltpu.sync_copy(
            out_v,
            out_hbm.at[pl.ds(row_base, _CHUNK_E), pl.ds(lane0, _D)],
        )

    _load_idx_fire(0, 0)
    for c in range(_NCHUNK):
        buf = c % 2
        if c + 1 < _NCHUNK:
            _load_idx_fire(c + 1, 1 - buf)
        _wait_rows(buf)
        _pool_store(c, buf)


_TBLK = 16384
_TGRID = _PLANE // _TBLK
_WBLKS = 1040000 // _TBLK


def _detile_body(w0_ref, w1_ref, w2_ref, w3_ref, x_ref):
    v = jnp.concatenate(
        [w0_ref[...], w1_ref[...], w2_ref[...], w3_ref[...]], axis=0
    )
    x_ref[...] = v.T


def _detile(tableT):
    specs = [
        pl.BlockSpec(
            (_D, _TBLK),
            lambda i, m=m: (0, jnp.minimum(i + _TGRID * m, _WBLKS)),
        )
        for m in range(4)
    ]
    return pl.pallas_call(
        _detile_body,
        out_shape=jax.ShapeDtypeStruct((_PLANE, 128), jnp.float32),
        grid=(_TGRID,),
        in_specs=specs,
        out_specs=pl.BlockSpec((_TBLK, 128), lambda i: (i, 0)),
    )(tableT, tableT, tableT, tableT)


def _untile_body(x_ref, o_ref):
    parts = [x_ref[:, _D * m:_D * (m + 1)].T for m in range(4)]
    o_ref[...] = jnp.concatenate(parts, axis=1)


def _untile(xout):
    return pl.pallas_call(
        _untile_body,
        out_shape=jax.ShapeDtypeStruct((_D, _B), jnp.float32),
        grid=(1,),
        in_specs=[pl.BlockSpec((_B // 4, 128), lambda i: (0, 0))],
        out_specs=pl.BlockSpec((_D, _B), lambda i: (0, 0)),
    )(xout)


def kernel(input_feat, table):
    table_lin = _detile(table.T).reshape(4 * _PLANE, _D)
    xout = _embed_pool(input_feat.T, table_lin)
    return _untile(xout).T
